# Initial kernel scaffold; baseline (speedup 1.0000x reference)
import jax
import jax.numpy as jnp
from jax.experimental import pallas as pl


def kernel(h, p, X_obs, M_obs, i_obs, w_prep, bias_prep, kernel, rec_kernel, gru_bias):
    raise NotImplementedError("write your pallas kernel here")


# speedup vs baseline: 1.0902x; 1.0902x over previous
import functools

import jax
import jax.numpy as jnp
from jax import lax
from jax.experimental import pallas as pl
from jax.experimental.pallas import tpu as pltpu

N_MEM = 100000
N_OBS = 16384
IN = 32
HID = 64
PH = 16
VAR_EPS = 1e-06

NU = 64
NBLK = (N_OBS // PH) // NU
OBS_BLK = PH * NU


def _dense_body(xp, mp, meanp, varp, hob, wpt, bpt,
                kz, kr, kh, rz, rr, rh,
                bxz, bxr, bxh, brz, brr, brh,
                out_h, out_loss, g_scr):
    x = xp[...]
    m = mp[...]
    mean = meanp[...]
    var = jnp.abs(varp[...]) + VAR_EPS
    err = (x - mean) / jnp.sqrt(var)

    loss_part = (0.5 * jnp.sum((err * err + jnp.log(var)) * m))[None, None]

    @pl.when(pl.program_id(0) == 0)
    def _init():
        out_loss[...] = jnp.zeros((1, 1), jnp.float32)

    out_loss[...] += loss_part

    w = wpt[...]
    b = bpt[...]
    for j in range(PH):
        sx = x[j][None, :, :]
        sm = mean[j][None, :, :]
        sv = var[j][None, :, :]
        se = err[j][None, :, :]
        a = (sx * w[:, 0:1, :] + sm * w[:, 1:2, :]
             + sv * w[:, 2:3, :] + se * w[:, 3:4, :] + b)
        a = jnp.maximum(a, 0.0) * m[j][None, :, :]
        g_scr[:, j * IN:(j + 1) * IN] = a.reshape(OBS_BLK, IN)

    g = g_scr[...]
    hf = hob[...].reshape(OBS_BLK, HID)
    xz = jnp.dot(g, kz[...], preferred_element_type=jnp.float32) + bxz[...]
    xr = jnp.dot(g, kr[...], preferred_element_type=jnp.float32) + bxr[...]
    xh = jnp.dot(g, kh[...], preferred_element_type=jnp.float32) + bxh[...]
    iz = jnp.dot(hf, rz[...], preferred_element_type=jnp.float32) + brz[...]
    ir = jnp.dot(hf, rr[...], preferred_element_type=jnp.float32) + brr[...]
    ih = jnp.dot(hf, rh[...], preferred_element_type=jnp.float32) + brh[...]
    z = jax.nn.sigmoid(xz + iz)
    r = jax.nn.sigmoid(xr + ir)
    hh = jnp.tanh(xh + r * ih)
    hn = z * hf + (1.0 - z) * hh
    out_h[...] = hn.reshape(PH, NU, HID)


def _dense_call(xp3, mp3, meanp3, varp3, hob3, wpt, bpt,
                kz, kr, kh, rz, rr, rh,
                bxz, bxr, bxh, brz, brr, brh, *, interpret=False):
    obs_spec = pl.BlockSpec((PH, NU, IN), lambda b: (0, b, 0))
    hid_spec = pl.BlockSpec((PH, NU, HID), lambda b: (0, b, 0))
    full = lambda shape: pl.BlockSpec(shape, lambda b: tuple(0 for _ in shape))
    return pl.pallas_call(
        _dense_body,
        grid=(NBLK,),
        in_specs=[obs_spec, obs_spec, obs_spec, obs_spec, hid_spec,
                  full((PH, 4, IN)), full((PH, 1, IN)),
                  full((PH * IN, HID)), full((PH * IN, HID)), full((PH * IN, HID)),
                  full((HID, HID)), full((HID, HID)), full((HID, HID)),
                  full((1, HID)), full((1, HID)), full((1, HID)),
                  full((1, HID)), full((1, HID)), full((1, HID))],
        out_specs=[hid_spec, pl.BlockSpec((1, 1), lambda b: (0, 0))],
        out_shape=[jax.ShapeDtypeStruct((PH, N_OBS // PH, HID), jnp.float32),
                   jax.ShapeDtypeStruct((1, 1), jnp.float32)],
        scratch_shapes=[pltpu.VMEM((OBS_BLK, PH * IN), jnp.float32)],
        interpret=interpret,
    )(xp3, mp3, meanp3, varp3, hob3, wpt, bpt,
      kz, kr, kh, rz, rr, rh, bxz, bxr, bxh, brz, brr, brh)


def _run(h, p, X_obs, M_obs, i_obs, w_prep, bias_prep, gru_kernel,
         rec_kernel, gru_bias, *, interpret=False):
    def permute(a):
        return (a.reshape(N_OBS // PH, PH, a.shape[-1])
                 .transpose(1, 0, 2).reshape(N_OBS, a.shape[-1]))

    Xp = permute(X_obs)
    Mp = permute(M_obs)
    i_obs_p = (i_obs.reshape(N_OBS // PH, PH).transpose(1, 0)
               .reshape(N_OBS))

    p_obs_p = jnp.take(p, i_obs_p, axis=0)
    h_obs = jnp.take(h, i_obs, axis=0)

    meanp = p_obs_p[:, :IN]
    varp = p_obs_p[:, IN:]

    xp3 = Xp.reshape(PH, N_OBS // PH, IN)
    mp3 = Mp.reshape(PH, N_OBS // PH, IN)
    meanp3 = meanp.reshape(PH, N_OBS // PH, IN)
    varp3 = varp.reshape(PH, N_OBS // PH, IN)
    hob3 = h_obs.reshape(PH, N_OBS // PH, HID)

    wpt = w_prep.transpose(2, 1, 0)
    bpt = bias_prep.transpose(1, 0).reshape(PH, 1, IN)
    kz = gru_kernel[:, 0 * HID:1 * HID]
    kr = gru_kernel[:, 1 * HID:2 * HID]
    kh = gru_kernel[:, 2 * HID:3 * HID]
    rz = rec_kernel[:, 0 * HID:1 * HID]
    rr = rec_kernel[:, 1 * HID:2 * HID]
    rh = rec_kernel[:, 2 * HID:3 * HID]
    bxz = gru_bias[0, 0 * HID:1 * HID].reshape(1, HID)
    bxr = gru_bias[0, 1 * HID:2 * HID].reshape(1, HID)
    bxh = gru_bias[0, 2 * HID:3 * HID].reshape(1, HID)
    brz = gru_bias[1, 0 * HID:1 * HID].reshape(1, HID)
    brr = gru_bias[1, 1 * HID:2 * HID].reshape(1, HID)
    brh = gru_bias[1, 2 * HID:3 * HID].reshape(1, HID)

    hn3, loss = _dense_call(xp3, mp3, meanp3, varp3, hob3, wpt, bpt,
                            kz, kr, kh, rz, rr, rh,
                            bxz, bxr, bxh, brz, brr, brh,
                            interpret=interpret)
    h_new = hn3.reshape(N_OBS, HID)

    h_out = h.at[i_obs].set(h_new)
    return h_out, loss.reshape(())


def kernel(h, p, X_obs, M_obs, i_obs, w_prep, bias_prep, kernel,
           rec_kernel, gru_bias):
    return _run(h, p, X_obs, M_obs, i_obs, w_prep, bias_prep, kernel,
                rec_kernel, gru_bias)
